# baseline (speedup 1.0000x reference)
import functools

import jax
import jax.numpy as jnp
from jax import lax
from jax.experimental import pallas as pl
from jax.experimental.pallas import tpu as pltpu
from jax.experimental.pallas import tpu_sc as plsc

CHUNK = 128
NBUF = 8


def _emb_lookup(idx3, weights, n_per_w, n_chunks, dim):
    info = plsc.get_sparse_core_info()
    num_cores = info.num_cores
    num_workers = info.num_cores * info.num_subcores
    n_total = num_workers * n_per_w
    mesh = plsc.VectorSubcoreMesh(core_axis_name="c", subcore_axis_name="s")

    @functools.partial(
        pl.kernel,
        out_type=jax.ShapeDtypeStruct((n_total, dim), jnp.float32),
        mesh=mesh,
        scratch_types=[
            pltpu.VMEM((n_chunks, CHUNK), jnp.int32),
            pltpu.VMEM((NBUF, CHUNK, dim), jnp.float32),
            pltpu.SemaphoreType.DMA((NBUF,)),
        ],
        compiler_params=pltpu.CompilerParams(use_tc_tiling_on_sc=False),
    )
    def emb(idx_hbm, table_hbm, out_hbm, idx_v, rows_v, gsem):
        wid = lax.axis_index("s") * num_cores + lax.axis_index("c")
        base = wid * n_per_w
        pltpu.sync_copy(idx_hbm.at[wid], idx_v)

        def gather(c, b):
            return pltpu.make_async_copy(
                table_hbm.at[idx_v.at[c]], rows_v.at[b], gsem.at[b]
            )

        for b in range(NBUF):
            gather(b, b).start()

        def group(g, _):
            for b in range(NBUF):
                c = g * NBUF + b
                gather(c, b).wait()
                pltpu.sync_copy(
                    rows_v.at[b], out_hbm.at[pl.ds(base + c * CHUNK, CHUNK)]
                )
                nxt = c + NBUF

                @pl.when(nxt < n_chunks)
                def _():
                    gather(nxt, b).start()

            return _

        lax.fori_loop(0, n_chunks // NBUF, group, None)

    return emb(idx3, weights)


def kernel(words, weights):
    batch, seq = words.shape
    _, dim = weights.shape
    n = batch * seq
    flat = words.reshape(n).astype(jnp.int32)

    info = plsc.get_sparse_core_info()
    num_workers = info.num_cores * info.num_subcores
    tile = num_workers * CHUNK
    n_pad = ((n + tile - 1) // tile) * tile
    if n_pad != n:
        flat = jnp.concatenate([flat, jnp.zeros(n_pad - n, jnp.int32)])
    n_per_w = n_pad // num_workers
    n_chunks = n_per_w // CHUNK

    idx3 = flat.reshape(num_workers, n_chunks, CHUNK)
    out = _emb_lookup(idx3, weights, n_per_w, n_chunks, dim)
    return out[:n].reshape(batch, seq, dim)


# speedup vs baseline: 1.0000x; 1.0000x over previous
import functools

import jax
import jax.numpy as jnp
from jax import lax
from jax.experimental import pallas as pl
from jax.experimental.pallas import tpu as pltpu
from jax.experimental.pallas import tpu_sc as plsc

CHUNK = 128
NBUF = 10


def _emb_lookup(idx3, weights, n_per_w, n_chunks, dim):
    info = plsc.get_sparse_core_info()
    num_cores = info.num_cores
    num_workers = info.num_cores * info.num_subcores
    n_total = num_workers * n_per_w
    mesh = plsc.VectorSubcoreMesh(core_axis_name="c", subcore_axis_name="s")

    @functools.partial(
        pl.kernel,
        out_type=jax.ShapeDtypeStruct((n_total, dim), jnp.float32),
        mesh=mesh,
        scratch_types=[
            pltpu.VMEM((n_chunks, CHUNK), jnp.int32),
            pltpu.VMEM((NBUF, CHUNK, dim), jnp.float32),
            pltpu.SemaphoreType.DMA((NBUF,)),
        ],
        compiler_params=pltpu.CompilerParams(use_tc_tiling_on_sc=False),
    )
    def emb(idx_hbm, table_hbm, out_hbm, idx_v, rows_v, gsem):
        wid = lax.axis_index("s") * num_cores + lax.axis_index("c")
        base = wid * n_per_w
        pltpu.sync_copy(idx_hbm.at[wid], idx_v)

        def gather(c, b):
            return pltpu.make_async_copy(
                table_hbm.at[idx_v.at[c]], rows_v.at[b], gsem.at[b]
            )

        for b in range(min(NBUF, n_chunks)):
            gather(b, b).start()

        def group(g, _):
            for b in range(NBUF):
                c = g * NBUF + b
                gather(c, b).wait()
                pltpu.sync_copy(
                    rows_v.at[b], out_hbm.at[pl.ds(base + c * CHUNK, CHUNK)]
                )
                nxt = c + NBUF

                @pl.when(nxt < n_chunks)
                def _():
                    gather(nxt, b).start()

            return _

        n_full = n_chunks // NBUF
        lax.fori_loop(0, n_full, group, None)
        for b in range(n_chunks % NBUF):
            c = n_full * NBUF + b
            gather(c, b).wait()
            pltpu.sync_copy(
                rows_v.at[b], out_hbm.at[pl.ds(base + c * CHUNK, CHUNK)]
            )

    return emb(idx3, weights)


def kernel(words, weights):
    batch, seq = words.shape
    _, dim = weights.shape
    n = batch * seq
    flat = words.reshape(n).astype(jnp.int32)

    info = plsc.get_sparse_core_info()
    num_workers = info.num_cores * info.num_subcores
    tile = num_workers * CHUNK
    n_pad = ((n + tile - 1) // tile) * tile
    if n_pad != n:
        flat = jnp.concatenate([flat, jnp.zeros(n_pad - n, jnp.int32)])
    n_per_w = n_pad // num_workers
    n_chunks = n_per_w // CHUNK

    idx3 = flat.reshape(num_workers, n_chunks, CHUNK)
    out = _emb_lookup(idx3, weights, n_per_w, n_chunks, dim)
    return out[:n].reshape(batch, seq, dim)
